# Initial kernel scaffold; baseline (speedup 1.0000x reference)
import jax
import jax.numpy as jnp
from jax.experimental import pallas as pl


def kernel(x, mask):
    raise NotImplementedError("write your pallas kernel here")


# speedup vs baseline: 1.1634x; 1.1634x over previous
import functools

import jax
import jax.numpy as jnp
from jax import lax
from jax.experimental import pallas as pl
from jax.experimental.pallas import tpu as pltpu
from jax.experimental.pallas import tpu_sc as plsc

_NUM_OBJS = 100000
_NUM_COLORS = 256
_N = _NUM_OBJS * _NUM_COLORS

_NUM_TILES = 32
_PER_TILE = _N // _NUM_TILES
_CHUNK = 16_000
_NBUF = 2
_VECS = _CHUNK // 16
_UNROLL = 8
_NCHUNK = _PER_TILE // _CHUNK
_BIG_I32 = 2**31 - 1

_MESH = plsc.VectorSubcoreMesh(core_axis_name="c", subcore_axis_name="s")


@functools.partial(
    pl.kernel,
    mesh=_MESH,
    out_type=(
        jax.ShapeDtypeStruct((_NUM_TILES, 16), jnp.float32),
        jax.ShapeDtypeStruct((_NUM_TILES, 16), jnp.int32),
    ),
    scratch_types=[
        pltpu.VMEM((_NBUF, _CHUNK), jnp.float32),
        pltpu.VMEM((_NBUF, _CHUNK), jnp.float32),
        pltpu.VMEM((16,), jnp.float32),
        pltpu.VMEM((16,), jnp.int32),
        pltpu.SemaphoreType.DMA((_NBUF,)),
        pltpu.SemaphoreType.DMA((_NBUF,)),
    ],
    compiler_params=pltpu.CompilerParams(needs_layout_passes=False),
)
def _scan(x_hbm, m_hbm, pmax_hbm, pidx_hbm, xb, mb, stage_f, stage_i, sx, sm):
    wid = lax.axis_index("c") * 16 + lax.axis_index("s")
    base = wid * _PER_TILE
    iota = lax.iota(jnp.int32, 16)

    def start_chunk(g, b):
        off = base + g * _CHUNK
        pltpu.async_copy(x_hbm.at[pl.ds(off, _CHUNK)], xb.at[b], sx.at[b])
        pltpu.async_copy(m_hbm.at[pl.ds(off, _CHUNK)], mb.at[b], sm.at[b])

    def wait_chunk(g, b):
        off = base + g * _CHUNK
        pltpu.make_async_copy(x_hbm.at[pl.ds(off, _CHUNK)], xb.at[b], sx.at[b]).wait()
        pltpu.make_async_copy(m_hbm.at[pl.ds(off, _CHUNK)], mb.at[b], sm.at[b]).wait()

    for b in range(_NBUF):
        start_chunk(b, b)

    ninf = jnp.full((16,), -jnp.inf, jnp.float32)
    zero_i = jnp.zeros((16,), jnp.int32)
    init = tuple([ninf] * _UNROLL) + tuple([zero_i] * _UNROLL)

    def outer(g2, carry):
        for b in range(_NBUF):
            g = g2 * _NBUF + b
            wait_chunk(g, b)
            gbase = base + g * _CHUNK

            def inner(j, c, b=b, gbase=gbase):
                vms = list(c[:_UNROLL])
                vis = list(c[_UNROLL:])
                jb = j * (16 * _UNROLL)
                ivec = gbase + jb + iota
                for u in range(_UNROLL):
                    o = jb + u * 16
                    v = xb[b, pl.ds(o, 16)] * mb[b, pl.ds(o, 16)]
                    cur = ivec + (u * 16)
                    p = v > vms[u]
                    vms[u] = jnp.where(p, v, vms[u])
                    vis[u] = jnp.where(p, cur, vis[u])
                return tuple(vms) + tuple(vis)

            carry = lax.fori_loop(0, _VECS // _UNROLL, inner, carry)

            @pl.when(g + _NBUF < _NCHUNK)
            def _(g=g, b=b):
                start_chunk(g + _NBUF, b)
        return carry

    acc = lax.fori_loop(0, _NCHUNK // _NBUF, outer, init)

    vm, vi = acc[0], acc[_UNROLL]
    for a in range(1, _UNROLL):
        v2, i2 = acc[a], acc[_UNROLL + a]
        p = (v2 > vm) | ((v2 == vm) & (i2 < vi))
        vm = jnp.where(p, v2, vm)
        vi = jnp.where(p, i2, vi)
    mx = jnp.max(vm)
    gidx = jnp.min(jnp.where(vm == mx, vi, _BIG_I32))

    stage_f[...] = jnp.full((16,), mx, jnp.float32)
    stage_i[...] = jnp.full((16,), gidx, jnp.int32)
    pltpu.sync_copy(stage_f, pmax_hbm.at[wid])
    pltpu.sync_copy(stage_i, pidx_hbm.at[wid])


@functools.partial(
    pl.kernel,
    mesh=_MESH,
    out_type=jax.ShapeDtypeStruct((_NUM_COLORS,), jnp.float32),
    scratch_types=[
        pltpu.VMEM((_NUM_TILES, 16), jnp.float32),
        pltpu.VMEM((_NUM_TILES, 16), jnp.int32),
        pltpu.VMEM((_NUM_COLORS,), jnp.float32),
        pltpu.VMEM((_NUM_COLORS,), jnp.float32),
    ],
    compiler_params=pltpu.CompilerParams(needs_layout_passes=False),
)
def _merge(x_hbm, pmax_hbm, pidx_hbm, out_hbm, pm, pi, child, out_v):
    cid = lax.axis_index("c")
    sid = lax.axis_index("s")

    pltpu.sync_copy(pmax_hbm, pm)
    pltpu.sync_copy(pidx_hbm, pi)
    iota = lax.iota(jnp.int32, 16)
    ones = jnp.full((16,), 1.0, jnp.float32)
    zeros = jnp.zeros((16,), jnp.float32)

    vm = pm[0, :]
    vi = pi[0, :]
    for w in range(1, _NUM_TILES):
        v2 = pm[w, :]
        i2 = pi[w, :]
        p = (v2 > vm) | ((v2 == vm) & (i2 < vi))
        vm = jnp.where(p, v2, vm)
        vi = jnp.where(p, i2, vi)
    mx = jnp.max(vm)
    gidx = jnp.min(jnp.where(vm == mx, vi, _BIG_I32))

    obj = gidx // _NUM_COLORS
    child_obj = (obj + 1) % _NUM_OBJS
    start = child_obj * _NUM_COLORS
    pltpu.sync_copy(x_hbm.at[pl.ds(start, _NUM_COLORS)], child)

    cm = child[pl.ds(0, 16)]
    ci = iota
    for j in range(1, _NUM_COLORS // 16):
        v = child[pl.ds(j * 16, 16)]
        cur = iota + (j * 16)
        p = v > cm
        cm = jnp.where(p, v, cm)
        ci = jnp.where(p, cur, ci)
    cmx = jnp.max(cm)
    cidx = jnp.min(jnp.where(cm == cmx, ci, _BIG_I32))
    cnew = (cidx + 1) % _NUM_COLORS

    for j in range(_NUM_COLORS // 16):
        cur = iota + (j * 16)
        out_v[pl.ds(j * 16, 16)] = jnp.where(cur == cnew, ones, zeros)

    @pl.when((cid == 0) & (sid == 0))
    def _():
        pltpu.sync_copy(out_v, out_hbm)


def kernel(x, mask):
    xf = x.reshape(_N)
    mf = mask.reshape(_N)
    pmax, pidx = _scan(xf, mf)
    return _merge(xf, pmax, pidx)


# speedup vs baseline: 1.8963x; 1.6299x over previous
import functools

import jax
import jax.numpy as jnp
from jax import lax
from jax.experimental import pallas as pl
from jax.experimental.pallas import tpu as pltpu
from jax.experimental.pallas import tpu_sc as plsc

_NUM_OBJS = 100000
_NUM_COLORS = 256
_N = _NUM_OBJS * _NUM_COLORS

_NUM_TILES = 32
_PER_TILE = _N // _NUM_TILES
_CHUNK = 6_400
_NBUF = 5
_VECS = _CHUNK // 16
_UNROLL = 8
_NCHUNK = _PER_TILE // _CHUNK
_BIG_I32 = 2**31 - 1

_MESH = plsc.VectorSubcoreMesh(core_axis_name="c", subcore_axis_name="s")


@functools.partial(
    pl.kernel,
    mesh=_MESH,
    out_type=(
        jax.ShapeDtypeStruct((_NUM_TILES, 16), jnp.float32),
        jax.ShapeDtypeStruct((_NUM_TILES, 16), jnp.int32),
    ),
    scratch_types=(
        [pltpu.VMEM((_CHUNK,), jnp.float32) for _ in range(2 * _NBUF)]
        + [
            pltpu.VMEM((16,), jnp.float32),
            pltpu.VMEM((16,), jnp.int32),
            pltpu.SemaphoreType.DMA((_NBUF,)),
            pltpu.SemaphoreType.DMA((_NBUF,)),
        ]
    ),
    compiler_params=pltpu.CompilerParams(needs_layout_passes=False),
)
def _scan(x_hbm, m_hbm, pmax_hbm, pidx_hbm, *_scr):
    xb = list(_scr[:_NBUF])
    mb = list(_scr[_NBUF:2 * _NBUF])
    stage_f, stage_i, sx, sm = _scr[2 * _NBUF:]
    wid = lax.axis_index("c") * 16 + lax.axis_index("s")
    base = wid * _PER_TILE
    iota = lax.iota(jnp.int32, 16)

    def start_chunk(g, b):
        off = base + g * _CHUNK
        pltpu.async_copy(x_hbm.at[pl.ds(off, _CHUNK)], xb[b], sx.at[b])
        pltpu.async_copy(m_hbm.at[pl.ds(off, _CHUNK)], mb[b], sm.at[b])

    def wait_chunk(g, b):
        off = base + g * _CHUNK
        pltpu.make_async_copy(x_hbm.at[pl.ds(off, _CHUNK)], xb[b], sx.at[b]).wait()
        pltpu.make_async_copy(m_hbm.at[pl.ds(off, _CHUNK)], mb[b], sm.at[b]).wait()

    for b in range(_NBUF - 1):
        start_chunk(b, b)

    ninf = jnp.full((16,), -jnp.inf, jnp.float32)
    zero_i = jnp.zeros((16,), jnp.int32)
    init = tuple([ninf] * _UNROLL) + tuple([zero_i] * _UNROLL)

    def outer(g2, carry):
        for b in range(_NBUF):
            g = g2 * _NBUF + b
            wait_chunk(g, b)

            @pl.when(g + _NBUF - 1 < _NCHUNK)
            def _(g=g, b=b):
                start_chunk(g + _NBUF - 1, (b + _NBUF - 1) % _NBUF)

            gbase = base + g * _CHUNK

            def inner(j, c, b=b, gbase=gbase):
                vms = list(c[:_UNROLL])
                vis = list(c[_UNROLL:])
                jb = j * (16 * _UNROLL)
                ivec = gbase + jb + iota
                for u in range(_UNROLL):
                    o = jb + u * 16
                    v = xb[b][pl.ds(o, 16)] * mb[b][pl.ds(o, 16)]
                    cur = ivec + (u * 16)
                    p = v > vms[u]
                    vms[u] = jnp.where(p, v, vms[u])
                    vis[u] = jnp.where(p, cur, vis[u])
                return tuple(vms) + tuple(vis)

            carry = lax.fori_loop(0, _VECS // _UNROLL, inner, carry)
        return carry

    acc = lax.fori_loop(0, _NCHUNK // _NBUF, outer, init)

    vm, vi = acc[0], acc[_UNROLL]
    for a in range(1, _UNROLL):
        v2, i2 = acc[a], acc[_UNROLL + a]
        p = (v2 > vm) | ((v2 == vm) & (i2 < vi))
        vm = jnp.where(p, v2, vm)
        vi = jnp.where(p, i2, vi)
    mx = jnp.max(vm)
    gidx = jnp.min(jnp.where(vm == mx, vi, _BIG_I32))

    stage_f[...] = jnp.full((16,), mx, jnp.float32)
    stage_i[...] = jnp.full((16,), gidx, jnp.int32)
    pltpu.sync_copy(stage_f, pmax_hbm.at[wid])
    pltpu.sync_copy(stage_i, pidx_hbm.at[wid])


@functools.partial(
    pl.kernel,
    mesh=_MESH,
    out_type=jax.ShapeDtypeStruct((_NUM_COLORS,), jnp.float32),
    scratch_types=[
        pltpu.VMEM((_NUM_TILES, 16), jnp.float32),
        pltpu.VMEM((_NUM_TILES, 16), jnp.int32),
        pltpu.VMEM((_NUM_COLORS,), jnp.float32),
        pltpu.VMEM((_NUM_COLORS,), jnp.float32),
    ],
    compiler_params=pltpu.CompilerParams(needs_layout_passes=False),
)
def _merge(x_hbm, pmax_hbm, pidx_hbm, out_hbm, pm, pi, child, out_v):
    cid = lax.axis_index("c")
    sid = lax.axis_index("s")

    pltpu.sync_copy(pmax_hbm, pm)
    pltpu.sync_copy(pidx_hbm, pi)
    iota = lax.iota(jnp.int32, 16)
    ones = jnp.full((16,), 1.0, jnp.float32)
    zeros = jnp.zeros((16,), jnp.float32)

    vm = pm[0, :]
    vi = pi[0, :]
    for w in range(1, _NUM_TILES):
        v2 = pm[w, :]
        i2 = pi[w, :]
        p = (v2 > vm) | ((v2 == vm) & (i2 < vi))
        vm = jnp.where(p, v2, vm)
        vi = jnp.where(p, i2, vi)
    mx = jnp.max(vm)
    gidx = jnp.min(jnp.where(vm == mx, vi, _BIG_I32))

    obj = gidx // _NUM_COLORS
    child_obj = (obj + 1) % _NUM_OBJS
    start = child_obj * _NUM_COLORS
    pltpu.sync_copy(x_hbm.at[pl.ds(start, _NUM_COLORS)], child)

    cm = child[pl.ds(0, 16)]
    ci = iota
    for j in range(1, _NUM_COLORS // 16):
        v = child[pl.ds(j * 16, 16)]
        cur = iota + (j * 16)
        p = v > cm
        cm = jnp.where(p, v, cm)
        ci = jnp.where(p, cur, ci)
    cmx = jnp.max(cm)
    cidx = jnp.min(jnp.where(cm == cmx, ci, _BIG_I32))
    cnew = (cidx + 1) % _NUM_COLORS

    for j in range(_NUM_COLORS // 16):
        cur = iota + (j * 16)
        out_v[pl.ds(j * 16, 16)] = jnp.where(cur == cnew, ones, zeros)

    @pl.when((cid == 0) & (sid == 0))
    def _():
        pltpu.sync_copy(out_v, out_hbm)


def kernel(x, mask):
    xf = x.reshape(_N)
    mf = mask.reshape(_N)
    pmax, pidx = _scan(xf, mf)
    return _merge(xf, pmax, pidx)
